# baseline (speedup 1.0000x reference)
import jax
import jax.numpy as jnp
from jax.experimental import pallas as pl


def _gat_collapse_kernel(ea_ref, w_ref, bt_ref, agg_ref, ee_ref):
    ea = ea_ref[...]
    w = w_ref[...]
    ee = jnp.sum(ea * w, axis=1, keepdims=True)
    ee_ref[...] = ee
    agg_ref[...] = jnp.maximum(ee + bt_ref[...], 0.0)


def kernel(x, edge_index, edge_attr, W, a, bias, edge_embedding_weight):
    E, _, D = edge_attr.shape
    O = bias.shape[0]
    K = a.shape[1]
    KO = K * O

    ea2 = edge_attr.reshape(E, D)
    w_row = edge_embedding_weight.reshape(1, D)
    bias_tiled = jnp.tile(bias, K).reshape(1, KO)

    BE = 128
    grid = (E // BE,)

    agg2, ee2 = pl.pallas_call(
        _gat_collapse_kernel,
        grid=grid,
        in_specs=[
            pl.BlockSpec((BE, D), lambda i: (i, 0)),
            pl.BlockSpec((1, D), lambda i: (0, 0)),
            pl.BlockSpec((1, KO), lambda i: (0, 0)),
        ],
        out_specs=[
            pl.BlockSpec((BE, KO), lambda i: (i, 0)),
            pl.BlockSpec((BE, 1), lambda i: (i, 0)),
        ],
        out_shape=[
            jax.ShapeDtypeStruct((E, KO), jnp.float32),
            jax.ShapeDtypeStruct((E, 1), jnp.float32),
        ],
    )(ea2, w_row, bias_tiled)

    aggregated = agg2.reshape(E, K, O)
    edge_embeddings = ee2.reshape(E, 1, 1)
    return (aggregated, edge_embeddings)
